# baseline (speedup 1.0000x reference)
import functools

import jax
import jax.numpy as jnp
from jax import lax
from jax.experimental import pallas as pl
from jax.experimental.pallas import tpu as pltpu
from jax.experimental.pallas import tpu_sc as plsc

_VOCAB = 1000000
_EMB = 64
_HID = 128
_OUT = 128
_BATCH = 16384
_BN_EPS = 1e-3

_B_TOT = 3 * _BATCH

_NC = 2
_NS = 16
_NW = _NC * _NS
_B_PER_W = _B_TOT // _NW
_CHUNK = 128
_N_CHUNK = _B_PER_W // _CHUNK
_NBUF = 4

_CB = 512
_TGRID = 980
_HH = _CB * _TGRID
_TAIL = 999936
_YCLAMP = _VOCAB // _CB - 1

_MLP_BLK = 2048


def _tr_body(x_ref, y_ref, eye_ref, o_ref):
    e = eye_ref[...]
    dims = (((0,), (0,)), ((), ()))
    xt = lax.dot_general(x_ref[...], e, dims,
                         preferred_element_type=jnp.float32)
    yt = lax.dot_general(y_ref[...], e, dims,
                         preferred_element_type=jnp.float32)
    o_ref[...] = jnp.concatenate([xt, yt], axis=1)


def _tc_pair_transpose(tt):
    return pl.pallas_call(
        _tr_body,
        grid=(_TGRID,),
        in_specs=[
            pl.BlockSpec((_EMB, _CB), lambda i: (0, i)),
            pl.BlockSpec(
                (_EMB, _CB),
                lambda i: (0, jnp.minimum(_TGRID + i, _YCLAMP))),
            pl.BlockSpec((_EMB, _EMB), lambda i: (0, 0)),
        ],
        out_specs=pl.BlockSpec((_CB, 2 * _EMB), lambda i: (i, 0)),
        out_shape=jax.ShapeDtypeStruct((_HH, 2 * _EMB), jnp.float32),
    )(tt, tt, jnp.eye(_EMB, dtype=jnp.float32))


def _sc_gather(t2, idx_pair):
    mesh = plsc.VectorSubcoreMesh(core_axis_name="c", subcore_axis_name="s")

    @functools.partial(
        pl.kernel,
        mesh=mesh,
        out_type=jax.ShapeDtypeStruct((_B_TOT, 2 * _EMB), jnp.float32),
        scratch_types=[
            pltpu.VMEM((_N_CHUNK, _CHUNK), jnp.int32),
            pltpu.VMEM((_NBUF, _CHUNK, 2 * _EMB), jnp.float32),
            pltpu.SemaphoreType.DMA((_NBUF,)),
            pltpu.SemaphoreType.DMA((_NBUF,)),
        ],
        compiler_params=pltpu.CompilerParams(needs_layout_passes=False),
    )
    def gather_kernel(t2_hbm, idx_hbm, out_hbm, idx_v, rows_v, gsem, wsem):
        wid = lax.axis_index("s") * _NC + lax.axis_index("c")
        base = wid * _B_PER_W
        pltpu.sync_copy(idx_hbm.at[wid], idx_v)

        def fire(j):
            return pltpu.async_copy(
                t2_hbm.at[idx_v.at[j]], rows_v.at[j % _NBUF],
                gsem.at[j % _NBUF])

        copies = [None] * _N_CHUNK
        writes = [None] * _N_CHUNK
        copies[0] = fire(0)
        for j in range(_N_CHUNK):
            b = j % _NBUF
            if j + 1 - _NBUF >= 0:
                writes[j + 1 - _NBUF].wait()
            if j + 1 < _N_CHUNK:
                copies[j + 1] = fire(j + 1)
            copies[j].wait()
            writes[j] = pltpu.async_copy(
                rows_v.at[b],
                out_hbm.at[pl.ds(base + j * _CHUNK, _CHUNK)],
                wsem.at[b])
        for j in range(_N_CHUNK - _NBUF + 1, _N_CHUNK):
            writes[j].wait()

    return gather_kernel(t2, idx_pair)


def _mlp_body(pr_ref, idx_ref, tail_ref, w1_ref, b1_ref, s_ref, t_ref,
              w2_ref, b2_ref, o_ref):
    pr = pr_ref[...]
    idx = idx_ref[...]
    x = jnp.where(idx >= _HH, pr[:, _EMB:], pr[:, :_EMB])
    t = idx - _TAIL
    oh = (t == lax.broadcasted_iota(jnp.int32, (_MLP_BLK, _EMB), 1))
    xfix = jnp.dot(oh.astype(jnp.float32), tail_ref[...],
                   preferred_element_type=jnp.float32)
    x = jnp.where(t >= 0, xfix, x)
    h = jnp.dot(x, w1_ref[...], preferred_element_type=jnp.float32)
    h = jnp.maximum(h + b1_ref[...], 0.0)
    h = h * s_ref[...] + t_ref[...]
    o_ref[...] = (
        jnp.dot(h, w2_ref[...], preferred_element_type=jnp.float32) + b2_ref[...]
    )


def _tc_mlp(pr, idxc, tail, W1, b1, scale, shift, W2, b2):
    grid = (_B_TOT // _MLP_BLK,)
    return pl.pallas_call(
        _mlp_body,
        grid=grid,
        in_specs=[
            pl.BlockSpec((_MLP_BLK, 2 * _EMB), lambda i: (i, 0)),
            pl.BlockSpec((_MLP_BLK, 1), lambda i: (i, 0)),
            pl.BlockSpec((_EMB, _EMB), lambda i: (0, 0)),
            pl.BlockSpec((_EMB, _HID), lambda i: (0, 0)),
            pl.BlockSpec((1, _HID), lambda i: (0, 0)),
            pl.BlockSpec((1, _HID), lambda i: (0, 0)),
            pl.BlockSpec((1, _HID), lambda i: (0, 0)),
            pl.BlockSpec((_HID, _OUT), lambda i: (0, 0)),
            pl.BlockSpec((1, _OUT), lambda i: (0, 0)),
        ],
        out_specs=pl.BlockSpec((_MLP_BLK, _OUT), lambda i: (i, 0)),
        out_shape=jax.ShapeDtypeStruct((_B_TOT, _OUT), jnp.float32),
    )(pr, idxc, tail, W1, b1, scale, shift, W2, b2)


def kernel(anchor, positive, negative, table, W1, b1, gamma, beta,
           moving_mean, moving_var, W2, b2):
    idx = jnp.concatenate([anchor, positive, negative]).astype(jnp.int32)
    idx_pair = jnp.where(idx >= _HH, idx - _HH, idx).reshape(
        _NW, _N_CHUNK, _CHUNK)
    tail = lax.slice(table, (_TAIL, 0), (_VOCAB, _EMB))

    t2 = _tc_pair_transpose(table.T)
    pr = _sc_gather(t2, idx_pair)

    scale = gamma * lax.rsqrt(moving_var + _BN_EPS)
    shift = beta - moving_mean * scale
    out = _tc_mlp(
        pr,
        idx.reshape(_B_TOT, 1),
        tail,
        W1,
        b1.reshape(1, _HID),
        scale.reshape(1, _HID),
        shift.reshape(1, _HID),
        W2,
        b2.reshape(1, _OUT),
    )
    return (out[:_BATCH], out[_BATCH:2 * _BATCH], out[2 * _BATCH:])


# speedup vs baseline: 1.8950x; 1.8950x over previous
import functools

import jax
import jax.numpy as jnp
from jax import lax
from jax.experimental import pallas as pl
from jax.experimental.pallas import tpu as pltpu
from jax.experimental.pallas import tpu_sc as plsc

_VOCAB = 1000000
_EMB = 64
_HID = 128
_OUT = 128
_BATCH = 16384
_BN_EPS = 1e-3

_B_TOT = 3 * _BATCH

_NC = 2
_NS = 16
_NW = _NC * _NS
_B_PER_W = _B_TOT // _NW
_CHUNK = 128
_N_CHUNK = _B_PER_W // _CHUNK
_NBUF = 4

_CB = 2048
_TGRID = 245
_HH = _CB * _TGRID
_TAIL = 999424
_TAILN = _VOCAB - _TAIL
_YCLAMP = _VOCAB // _CB - 1

_MLP_BLK = 2048


def _tr_body(x_ref, y_ref, eye_ref, o_ref):
    e = eye_ref[...]
    dims = (((0,), (0,)), ((), ()))
    xt = lax.dot_general(x_ref[...], e, dims,
                         preferred_element_type=jnp.float32)
    yt = lax.dot_general(y_ref[...], e, dims,
                         preferred_element_type=jnp.float32)
    o_ref[...] = jnp.concatenate([xt, yt], axis=1)


def _tc_pair_transpose(tt):
    return pl.pallas_call(
        _tr_body,
        grid=(_TGRID,),
        in_specs=[
            pl.BlockSpec((_EMB, _CB), lambda i: (0, i)),
            pl.BlockSpec(
                (_EMB, _CB),
                lambda i: (0, jnp.minimum(_TGRID + i, _YCLAMP))),
            pl.BlockSpec((_EMB, _EMB), lambda i: (0, 0)),
        ],
        out_specs=pl.BlockSpec((_CB, 2 * _EMB), lambda i: (i, 0)),
        out_shape=jax.ShapeDtypeStruct((_HH, 2 * _EMB), jnp.float32),
    )(tt, tt, jnp.eye(_EMB, dtype=jnp.float32))


def _sc_gather(t2, idx_pair):
    mesh = plsc.VectorSubcoreMesh(core_axis_name="c", subcore_axis_name="s")

    @functools.partial(
        pl.kernel,
        mesh=mesh,
        out_type=jax.ShapeDtypeStruct((_B_TOT, 2 * _EMB), jnp.float32),
        scratch_types=[
            pltpu.VMEM((_N_CHUNK, _CHUNK), jnp.int32),
            pltpu.VMEM((_NBUF, _CHUNK, 2 * _EMB), jnp.float32),
            pltpu.SemaphoreType.DMA((_NBUF,)),
            pltpu.SemaphoreType.DMA((_NBUF,)),
        ],
        compiler_params=pltpu.CompilerParams(needs_layout_passes=False),
    )
    def gather_kernel(t2_hbm, idx_hbm, out_hbm, idx_v, rows_v, gsem, wsem):
        wid = lax.axis_index("s") * _NC + lax.axis_index("c")
        base = wid * _B_PER_W
        pltpu.sync_copy(idx_hbm.at[wid], idx_v)

        def fire(j):
            return pltpu.async_copy(
                t2_hbm.at[idx_v.at[j]], rows_v.at[j % _NBUF],
                gsem.at[j % _NBUF])

        copies = [None] * _N_CHUNK
        writes = [None] * _N_CHUNK
        copies[0] = fire(0)
        for j in range(_N_CHUNK):
            b = j % _NBUF
            if j + 1 - _NBUF >= 0:
                writes[j + 1 - _NBUF].wait()
            if j + 1 < _N_CHUNK:
                copies[j + 1] = fire(j + 1)
            copies[j].wait()
            writes[j] = pltpu.async_copy(
                rows_v.at[b],
                out_hbm.at[pl.ds(base + j * _CHUNK, _CHUNK)],
                wsem.at[b])
        for j in range(_N_CHUNK - _NBUF + 1, _N_CHUNK):
            writes[j].wait()

    return gather_kernel(t2, idx_pair)


def _mlp_body(pr_ref, idx_ref, tail_ref, w1_ref, b1_ref, s_ref, t_ref,
              w2_ref, b2_ref, o_ref):
    pr = pr_ref[...]
    idx = idx_ref[...]
    x = jnp.where(idx >= _HH, pr[:, _EMB:], pr[:, :_EMB])
    t = idx - _TAIL
    oh = (t == lax.broadcasted_iota(jnp.int32, (_MLP_BLK, _TAILN), 1))
    xfix = jnp.dot(oh.astype(jnp.float32), tail_ref[...],
                   preferred_element_type=jnp.float32)
    x = jnp.where(t >= 0, xfix, x)
    h = jnp.dot(x, w1_ref[...], preferred_element_type=jnp.float32)
    h = jnp.maximum(h + b1_ref[...], 0.0)
    h = h * s_ref[...] + t_ref[...]
    o_ref[...] = (
        jnp.dot(h, w2_ref[...], preferred_element_type=jnp.float32) + b2_ref[...]
    )


def _tc_mlp(pr, idxc, tail, W1, b1, scale, shift, W2, b2):
    grid = (_B_TOT // _MLP_BLK,)
    return pl.pallas_call(
        _mlp_body,
        grid=grid,
        in_specs=[
            pl.BlockSpec((_MLP_BLK, 2 * _EMB), lambda i: (i, 0)),
            pl.BlockSpec((_MLP_BLK, 1), lambda i: (i, 0)),
            pl.BlockSpec((_TAILN, _EMB), lambda i: (0, 0)),
            pl.BlockSpec((_EMB, _HID), lambda i: (0, 0)),
            pl.BlockSpec((1, _HID), lambda i: (0, 0)),
            pl.BlockSpec((1, _HID), lambda i: (0, 0)),
            pl.BlockSpec((1, _HID), lambda i: (0, 0)),
            pl.BlockSpec((_HID, _OUT), lambda i: (0, 0)),
            pl.BlockSpec((1, _OUT), lambda i: (0, 0)),
        ],
        out_specs=pl.BlockSpec((_MLP_BLK, _OUT), lambda i: (i, 0)),
        out_shape=jax.ShapeDtypeStruct((_B_TOT, _OUT), jnp.float32),
    )(pr, idxc, tail, W1, b1, scale, shift, W2, b2)


def kernel(anchor, positive, negative, table, W1, b1, gamma, beta,
           moving_mean, moving_var, W2, b2):
    idx = jnp.concatenate([anchor, positive, negative]).astype(jnp.int32)
    idx_pair = jnp.where(idx >= _HH, idx - _HH, idx).reshape(
        _NW, _N_CHUNK, _CHUNK)
    tail = lax.slice(table, (_TAIL, 0), (_VOCAB, _EMB))

    t2 = _tc_pair_transpose(table.T)
    pr = _sc_gather(t2, idx_pair)

    scale = gamma * lax.rsqrt(moving_var + _BN_EPS)
    shift = beta - moving_mean * scale
    out = _tc_mlp(
        pr,
        idx.reshape(_B_TOT, 1),
        tail,
        W1,
        b1.reshape(1, _HID),
        scale.reshape(1, _HID),
        shift.reshape(1, _HID),
        W2,
        b2.reshape(1, _OUT),
    )
    return (out[:_BATCH], out[_BATCH:2 * _BATCH], out[2 * _BATCH:])
